# baseline (speedup 1.0000x reference)
import functools

import jax
import jax.numpy as jnp
from jax import lax
from jax.experimental import pallas as pl
from jax.experimental.pallas import tpu as pltpu
from jax.experimental.pallas import tpu_sc as plsc

_N = 20000
_C = 81
_D = 256
_K = 1000
_SCORE_COL = 80

_NT = 16
_NP = 1280
_VR = _NP // 16
_PAD_EV = 65535

_GT = 25
_GR = 40
_LGW = _GR * _C
_CP = _C
_LGCH = 26
_BBW = _GR * 4
_BBCH = 2


def _build_sc_kernel():
    mesh = plsc.VectorSubcoreMesh(core_axis_name="c", subcore_axis_name="s")

    @functools.partial(
        pl.kernel,
        mesh=mesh,
        compiler_params=pltpu.CompilerParams(
            needs_layout_passes=False, use_tc_tiling_on_sc=False),
        out_type=(
            jax.ShapeDtypeStruct((_K * 4,), jnp.float32),
            jax.ShapeDtypeStruct((_K * _C,), jnp.float32),
            jax.ShapeDtypeStruct((_K, _D), jnp.float32),
        ),
        scratch_types=[
            pltpu.VMEM((_NP,), jnp.int32),
            pltpu.VMEM((_NP,), jnp.float32),
            pltpu.VMEM((_NP,), jnp.int32),
            pltpu.VMEM((256,), jnp.int32),
            pltpu.VMEM((4096,), jnp.int32),
            pltpu.VMEM((_NP + 16,), jnp.int32),
            pltpu.VMEM((_NP + 16,), jnp.int32),
            pltpu.VMEM((16,), jnp.int32),
            pltpu.VMEM((256,), jnp.int32),
            pltpu.VMEM((10, 128), jnp.int32),
            pltpu.VMEM((1024,), jnp.int32),
            pltpu.VMEM((1024,), jnp.int32),
            pltpu.VMEM((1, 64), jnp.int32),
            pltpu.VMEM((64,), jnp.int32),
            pltpu.VMEM((48,), jnp.int32),
            pltpu.VMEM((_GR, _D), jnp.float32),
            pltpu.VMEM((_LGCH * 128 + 16,), jnp.int32),
            pltpu.VMEM((_LGCH * 128,), jnp.float32),
            pltpu.VMEM((_BBCH * 128,), jnp.int32),
            pltpu.VMEM((_BBCH * 128,), jnp.float32),
            pltpu.VMEM_SHARED((4096,), jnp.int32),
            pltpu.VMEM_SHARED((256,), jnp.int32),
            pltpu.VMEM_SHARED((1040,), jnp.int32),
            pltpu.VMEM_SHARED((1040,), jnp.int32),
            pltpu.VMEM_SHARED((1040,), jnp.int32),
            pltpu.SemaphoreType.DMA,
            pltpu.SemaphoreType.DMA,
            pltpu.SemaphoreType.DMA,
        ],
    )
    def sc_kernel(pf, bbf, lgf, emb, ob, ol, oe,
                  sidx_v, sval_v, bits_v, hist_v, hall_v, candb_v, cande_v,
                  my_v, cnts_v, pos2_v, call_b, call_e, rpos_v, rval_v,
                  ridx_v, em_rows, lgi_v, lgd_v, bbi_v, bbd_v,
                  sh_hist, sh_cnt, sh_candb, sh_cande, sh_sorted,
                  sem_a, sem_b, sem_c):
        cid = lax.axis_index("c")
        tid = lax.axis_index("s")
        wid = cid * _NT + tid
        iota = lax.iota(jnp.int32, 16)
        ones = jnp.full((16,), 1, jnp.int32)
        base = tid * _NP

        def p0_idx(j, _):
            g = base + j * 16 + iota
            sidx_v[pl.ds(j * 16, 16)] = jnp.where(
                g < _N, g * _CP + _SCORE_COL, 0)
            return 0

        lax.fori_loop(0, _VR, p0_idx, 0)
        cps = [pltpu.async_copy(pf.at[sidx_v.at[pl.ds(128 * c, 128)]],
                                sval_v.at[pl.ds(128 * c, 128)], sem_a)
               for c in range(_NP // 128)]
        for cp in cps:
            cp.wait()

        def p0_bits(j, _):
            s = sval_v[pl.ds(j * 16, 16)]
            b = plsc.bitcast(s, jnp.int32)
            g = base + j * 16 + iota
            bits_v[pl.ds(j * 16, 16)] = jnp.where(g < _N, b, 0)
            return 0

        lax.fori_loop(0, _VR, p0_bits, 0)

        R = jnp.int32(_K)
        fp = jnp.int32(0)
        d5 = jnp.int32(0)
        istar = jnp.int32(0)
        bstar = jnp.int32(0)

        for p in range(6):
            def zero_h(t, _):
                hist_v[pl.ds(t * 16, 16)] = iota * 0
                return 0

            lax.fori_loop(0, 16, zero_h, 0)

            fpb = fp

            def hist_body(j, _, p=p, fpb=fpb, R=R, d5=d5, bstar=bstar):
                b = bits_v[pl.ds(j * 16, 16)]
                g = base + j * 16 + iota
                ev = jnp.where(g < _N, g, _PAD_EV)
                if p == 0:
                    cand = jnp.full((16,), True)
                    dig = (b >> 24) & 255
                elif p == 1:
                    cand = (b >> 24) == jnp.full((16,), fpb >> 24)
                    dig = (b >> 16) & 255
                elif p == 2:
                    cand = (b >> 16) == jnp.full((16,), fpb >> 16)
                    dig = (b >> 8) & 255
                elif p == 3:
                    cand = (b >> 8) == jnp.full((16,), fpb >> 8)
                    dig = b & 255
                elif p == 4:
                    cand = b == jnp.full((16,), bstar)
                    dig = (ev >> 8) & 255
                else:
                    cand = ((b == jnp.full((16,), bstar))
                            & ((ev >> 8) == jnp.full((16,), d5)))
                    dig = ev & 255
                plsc.addupdate_scatter(hist_v, [dig], ones, mask=cand)
                return 0

            lax.fori_loop(0, _VR, hist_body, 0)

            plsc.subcore_barrier()
            pltpu.sync_copy(hist_v, sh_hist.at[pl.ds(tid * 256, 256)])
            plsc.subcore_barrier()
            pltpu.sync_copy(sh_hist, hall_v)

            hs = []
            for v in range(16):
                def red_body(r, acc, v=v):
                    return acc + hall_v[pl.ds(r * 256 + v * 16, 16)]

                hs.append(lax.fori_loop(0, 16, red_body, iota * 0))

            tv = [jnp.sum(h) for h in hs]
            if p < 4:
                suf = [jnp.int32(0)] * 17
                for v in range(15, -1, -1):
                    suf[v] = suf[v + 1] + tv[v]
                dstar = jnp.int32(-1)
                slanes = []
                for v in range(16):
                    s_lane = (jnp.full((16,), suf[v + 1])
                              + lax.rev(plsc.cumsum(lax.rev(hs[v], (0,))),
                                        (0,)))
                    slanes.append(s_lane)
                    digv = iota + 16 * v
                    cd = jnp.where(s_lane >= jnp.full((16,), R), digv, -1)
                    dstar = jnp.maximum(dstar, jnp.max(cd))
                hd = jnp.int32(0)
                sd = jnp.int32(0)
                for v in range(16):
                    selv = (iota + 16 * v) == jnp.full((16,), dstar)
                    hd = hd + jnp.sum(jnp.where(selv, hs[v], 0))
                    sd = sd + jnp.sum(jnp.where(selv, slanes[v], 0))
                R = R - (sd - hd)
                fp = fp | (dstar << (24 - 8 * p))
                if p == 3:
                    bstar = fp
            else:
                pre = [jnp.int32(0)] * 17
                for v in range(16):
                    pre[v + 1] = pre[v] + tv[v]
                dstar = jnp.int32(9999)
                planes = []
                for v in range(16):
                    p_lane = jnp.full((16,), pre[v]) + plsc.cumsum(hs[v])
                    planes.append(p_lane)
                    digv = iota + 16 * v
                    cd = jnp.where(p_lane >= jnp.full((16,), R), digv, 9999)
                    dstar = jnp.minimum(dstar, jnp.min(cd))
                hd = jnp.int32(0)
                pd = jnp.int32(0)
                for v in range(16):
                    selv = (iota + 16 * v) == jnp.full((16,), dstar)
                    hd = hd + jnp.sum(jnp.where(selv, hs[v], 0))
                    pd = pd + jnp.sum(jnp.where(selv, planes[v], 0))
                R = R - (pd - hd)
                if p == 4:
                    d5 = dstar
                else:
                    istar = (d5 << 8) | dstar

        bst = jnp.full((16,), bstar)
        ist = jnp.full((16,), istar)

        def compact_body(j, m):
            b = bits_v[pl.ds(j * 16, 16)]
            g = base + j * 16 + iota
            ev = jnp.where(g < _N, g, _PAD_EV)
            sel = (b > bst) | ((b == bst) & (ev <= ist))
            plsc.store_compressed(candb_v.at[pl.ds(m, 16)], b, mask=sel)
            plsc.store_compressed(cande_v.at[pl.ds(m, 16)], ev, mask=sel)
            return m + jnp.sum(sel.astype(jnp.int32))

        m = lax.fori_loop(0, _VR, compact_body, jnp.int32(0))

        my_v[...] = jnp.full((16,), m, jnp.int32)
        pltpu.sync_copy(my_v, sh_cnt.at[pl.ds(tid * 16, 16)])
        plsc.subcore_barrier()
        pltpu.sync_copy(sh_cnt, cnts_v)
        cnts = plsc.load_gather(cnts_v, [iota * 16])
        off = jnp.sum(jnp.where(iota < tid, cnts, 0))

        for c in range(10):
            for v in range(8):
                j = 128 * c + 16 * v + iota
                pos = jnp.where(j < jnp.full((16,), m), off + j, 1024 + tid)
                pos2_v[c, pl.ds(16 * v, 16)] = pos
        cps = []
        for c in range(10):
            cps.append(pltpu.async_copy(
                candb_v.at[pl.ds(128 * c, 128)],
                sh_candb.at[pos2_v.at[c]], sem_b))
            cps.append(pltpu.async_copy(
                cande_v.at[pl.ds(128 * c, 128)],
                sh_cande.at[pos2_v.at[c]], sem_b))
        for cp in cps:
            cp.wait()
        plsc.subcore_barrier()

        pltpu.sync_copy(sh_candb.at[pl.ds(0, 1024)], call_b)
        pltpu.sync_copy(sh_cande.at[pl.ds(0, 1024)], call_e)
        v62 = call_b[pl.ds(992, 16)]
        call_b[pl.ds(992, 16)] = jnp.where(iota < 8, v62, -1)
        call_b[pl.ds(1008, 16)] = jnp.full((16,), -1, jnp.int32)

        q0 = tid * 64
        for o in range(4):
            a_b = call_b[pl.ds(q0 + 16 * o, 16)]
            a_e = call_e[pl.ds(q0 + 16 * o, 16)]

            def pair_body(j, acc, a_b=a_b, a_e=a_e):
                b_b = call_b[pl.ds(j * 16, 16)]
                b_e = call_e[pl.ds(j * 16, 16)]
                for r in range(16):
                    ridx = (iota + r) % 16
                    bb = b_b[ridx]
                    be = b_e[ridx]
                    gt = bb > a_b
                    tie = (bb == a_b) & (be < a_e)
                    acc = acc + jnp.where(gt | tie, 1, 0)
                return acc

            acc = lax.fori_loop(0, 64, pair_body, iota * 0)
            q = q0 + 16 * o + iota
            rpos_v[0, pl.ds(16 * o, 16)] = jnp.where(
                q < _K, acc, 1024 + tid)
            rval_v[pl.ds(16 * o, 16)] = a_e
        pltpu.async_copy(rval_v, sh_sorted.at[rpos_v.at[0]], sem_c).wait()
        plsc.subcore_barrier()

        @pl.when(wid < _GT)
        def _():
            pltpu.sync_copy(sh_sorted.at[pl.ds(wid * _GR, _GR)],
                            ridx_v.at[pl.ds(0, _GR)])
            cpe = pltpu.async_copy(emb.at[ridx_v.at[pl.ds(0, _GR)]],
                                   em_rows, sem_a)

            def lg_idx(t, _):
                fpos = t * 16 + iota
                r = fpos // _C
                col = fpos - r * _C
                rid = plsc.load_gather(ridx_v, [jnp.minimum(r, 47)])
                lgi_v[pl.ds(t * 16, 16)] = jnp.where(
                    fpos < _LGW, rid * _CP + col, 0)
                return 0

            lax.fori_loop(0, (_LGCH * 128) // 16, lg_idx, 0)

            def bb_idx(t, _):
                fpos = t * 16 + iota
                r = fpos >> 2
                col = fpos & 3
                rid = plsc.load_gather(ridx_v, [jnp.minimum(r, 47)])
                bbi_v[pl.ds(t * 16, 16)] = jnp.where(
                    fpos < _BBW, rid * 4 + col, 0)
                return 0

            lax.fori_loop(0, (_BBCH * 128) // 16, bb_idx, 0)

            cpl = [pltpu.async_copy(lgf.at[lgi_v.at[pl.ds(128 * c, 128)]],
                                    lgd_v.at[pl.ds(128 * c, 128)], sem_b)
                   for c in range(_LGCH)]
            cpb = [pltpu.async_copy(bbf.at[bbi_v.at[pl.ds(128 * c, 128)]],
                                    bbd_v.at[pl.ds(128 * c, 128)], sem_c)
                   for c in range(_BBCH)]
            for cp in cpl:
                cp.wait()
            for cp in cpb:
                cp.wait()
            cpe.wait()
            pltpu.sync_copy(em_rows, oe.at[pl.ds(wid * _GR, _GR)])
            pltpu.sync_copy(lgd_v.at[pl.ds(0, _LGW)],
                            ol.at[pl.ds(wid * _LGW, _LGW)])
            pltpu.sync_copy(bbd_v.at[pl.ds(0, _BBW)],
                            ob.at[pl.ds(wid * _BBW, _BBW)])

    return sc_kernel


def kernel(cls_probs, bboxes_cxcywh, cls_logits, embeds, image_size):
    sc = _build_sc_kernel()
    ob, ol, oe = sc(
        cls_probs.reshape(-1),
        bboxes_cxcywh.reshape(-1),
        cls_logits.reshape(-1),
        embeds,
    )
    return (ob.reshape(_K, 4), ol.reshape(_K, _C), oe, image_size)


# speedup vs baseline: 1.0947x; 1.0947x over previous
import functools

import jax
import jax.numpy as jnp
from jax import lax
from jax.experimental import pallas as pl
from jax.experimental.pallas import tpu as pltpu
from jax.experimental.pallas import tpu_sc as plsc

_N = 20000
_C = 81
_D = 256
_K = 1000
_SCORE_COL = 80

_NT = 16
_NP = 1280
_VR = _NP // 16
_PAD_EV = 65535

_GT = 25
_GR = 40
_LGW = _GR * _C
_CP = 128
_LGCH = 26
_BBW = _GR * 4
_BBCH = 2


def _build_sc_kernel():
    mesh = plsc.VectorSubcoreMesh(core_axis_name="c", subcore_axis_name="s")

    @functools.partial(
        pl.kernel,
        mesh=mesh,
        compiler_params=pltpu.CompilerParams(needs_layout_passes=False),
        out_type=(
            jax.ShapeDtypeStruct((_K * 4,), jnp.float32),
            jax.ShapeDtypeStruct((_K * _C,), jnp.float32),
            jax.ShapeDtypeStruct((_K, _D), jnp.float32),
        ),
        scratch_types=[
            pltpu.VMEM((_NP,), jnp.int32),
            pltpu.VMEM((_NP,), jnp.float32),
            pltpu.VMEM((_NP,), jnp.int32),
            pltpu.VMEM((256,), jnp.int32),
            pltpu.VMEM((4096,), jnp.int32),
            pltpu.VMEM((_NP + 16,), jnp.int32),
            pltpu.VMEM((_NP + 16,), jnp.int32),
            pltpu.VMEM((16,), jnp.int32),
            pltpu.VMEM((256,), jnp.int32),
            pltpu.VMEM((10, 128), jnp.int32),
            pltpu.VMEM((1024,), jnp.int32),
            pltpu.VMEM((1024,), jnp.int32),
            pltpu.VMEM((1, 64), jnp.int32),
            pltpu.VMEM((64,), jnp.int32),
            pltpu.VMEM((48,), jnp.int32),
            pltpu.VMEM((_GR, _D), jnp.float32),
            pltpu.VMEM((_LGCH * 128 + 16,), jnp.int32),
            pltpu.VMEM((_LGCH * 128,), jnp.float32),
            pltpu.VMEM((_BBCH * 128,), jnp.int32),
            pltpu.VMEM((_BBCH * 128,), jnp.float32),
            pltpu.VMEM_SHARED((4096,), jnp.int32),
            pltpu.VMEM_SHARED((256,), jnp.int32),
            pltpu.VMEM_SHARED((1040,), jnp.int32),
            pltpu.VMEM_SHARED((1040,), jnp.int32),
            pltpu.VMEM_SHARED((1040,), jnp.int32),
            pltpu.SemaphoreType.DMA,
            pltpu.SemaphoreType.DMA,
            pltpu.SemaphoreType.DMA,
        ],
    )
    def sc_kernel(pf, bbf, lgf, emb, ob, ol, oe,
                  sidx_v, sval_v, bits_v, hist_v, hall_v, candb_v, cande_v,
                  my_v, cnts_v, pos2_v, call_b, call_e, rpos_v, rval_v,
                  ridx_v, em_rows, lgi_v, lgd_v, bbi_v, bbd_v,
                  sh_hist, sh_cnt, sh_candb, sh_cande, sh_sorted,
                  sem_a, sem_b, sem_c):
        cid = lax.axis_index("c")
        tid = lax.axis_index("s")
        wid = cid * _NT + tid
        iota = lax.iota(jnp.int32, 16)
        ones = jnp.full((16,), 1, jnp.int32)
        base = tid * _NP

        def p0_idx(j, _):
            g = base + j * 16 + iota
            sidx_v[pl.ds(j * 16, 16)] = jnp.where(
                g < _N, g * _CP + _SCORE_COL, 0)
            return 0

        lax.fori_loop(0, _VR, p0_idx, 0)
        cps = [pltpu.async_copy(pf.at[sidx_v.at[pl.ds(128 * c, 128)]],
                                sval_v.at[pl.ds(128 * c, 128)], sem_a)
               for c in range(_NP // 128)]
        for cp in cps:
            cp.wait()

        def p0_bits(j, _):
            s = sval_v[pl.ds(j * 16, 16)]
            b = plsc.bitcast(s, jnp.int32)
            g = base + j * 16 + iota
            bits_v[pl.ds(j * 16, 16)] = jnp.where(g < _N, b, 0)
            return 0

        lax.fori_loop(0, _VR, p0_bits, 0)

        R = jnp.int32(_K)
        fp = jnp.int32(0)
        d5 = jnp.int32(0)
        istar = jnp.int32(0)
        bstar = jnp.int32(0)

        for p in range(6):
            def zero_h(t, _):
                hist_v[pl.ds(t * 16, 16)] = iota * 0
                return 0

            lax.fori_loop(0, 16, zero_h, 0)

            fpb = fp

            def hist_body(j, _, p=p, fpb=fpb, R=R, d5=d5, bstar=bstar):
                b = bits_v[pl.ds(j * 16, 16)]
                g = base + j * 16 + iota
                ev = jnp.where(g < _N, g, _PAD_EV)
                if p == 0:
                    cand = jnp.full((16,), True)
                    dig = (b >> 24) & 255
                elif p == 1:
                    cand = (b >> 24) == jnp.full((16,), fpb >> 24)
                    dig = (b >> 16) & 255
                elif p == 2:
                    cand = (b >> 16) == jnp.full((16,), fpb >> 16)
                    dig = (b >> 8) & 255
                elif p == 3:
                    cand = (b >> 8) == jnp.full((16,), fpb >> 8)
                    dig = b & 255
                elif p == 4:
                    cand = b == jnp.full((16,), bstar)
                    dig = (ev >> 8) & 255
                else:
                    cand = ((b == jnp.full((16,), bstar))
                            & ((ev >> 8) == jnp.full((16,), d5)))
                    dig = ev & 255
                plsc.addupdate_scatter(hist_v, [dig], ones, mask=cand)
                return 0

            lax.fori_loop(0, _VR, hist_body, 0)

            plsc.subcore_barrier()
            pltpu.sync_copy(hist_v, sh_hist.at[pl.ds(tid * 256, 256)])
            plsc.subcore_barrier()
            pltpu.sync_copy(sh_hist, hall_v)

            hs = []
            for v in range(16):
                def red_body(r, acc, v=v):
                    return acc + hall_v[pl.ds(r * 256 + v * 16, 16)]

                hs.append(lax.fori_loop(0, 16, red_body, iota * 0))

            tv = [jnp.sum(h) for h in hs]
            if p < 4:
                suf = [jnp.int32(0)] * 17
                for v in range(15, -1, -1):
                    suf[v] = suf[v + 1] + tv[v]
                dstar = jnp.int32(-1)
                slanes = []
                for v in range(16):
                    s_lane = (jnp.full((16,), suf[v + 1])
                              + lax.rev(plsc.cumsum(lax.rev(hs[v], (0,))),
                                        (0,)))
                    slanes.append(s_lane)
                    digv = iota + 16 * v
                    cd = jnp.where(s_lane >= jnp.full((16,), R), digv, -1)
                    dstar = jnp.maximum(dstar, jnp.max(cd))
                hd = jnp.int32(0)
                sd = jnp.int32(0)
                for v in range(16):
                    selv = (iota + 16 * v) == jnp.full((16,), dstar)
                    hd = hd + jnp.sum(jnp.where(selv, hs[v], 0))
                    sd = sd + jnp.sum(jnp.where(selv, slanes[v], 0))
                R = R - (sd - hd)
                fp = fp | (dstar << (24 - 8 * p))
                if p == 3:
                    bstar = fp
            else:
                pre = [jnp.int32(0)] * 17
                for v in range(16):
                    pre[v + 1] = pre[v] + tv[v]
                dstar = jnp.int32(9999)
                planes = []
                for v in range(16):
                    p_lane = jnp.full((16,), pre[v]) + plsc.cumsum(hs[v])
                    planes.append(p_lane)
                    digv = iota + 16 * v
                    cd = jnp.where(p_lane >= jnp.full((16,), R), digv, 9999)
                    dstar = jnp.minimum(dstar, jnp.min(cd))
                hd = jnp.int32(0)
                pd = jnp.int32(0)
                for v in range(16):
                    selv = (iota + 16 * v) == jnp.full((16,), dstar)
                    hd = hd + jnp.sum(jnp.where(selv, hs[v], 0))
                    pd = pd + jnp.sum(jnp.where(selv, planes[v], 0))
                R = R - (pd - hd)
                if p == 4:
                    d5 = dstar
                else:
                    istar = (d5 << 8) | dstar

        bst = jnp.full((16,), bstar)
        ist = jnp.full((16,), istar)

        def compact_body(j, m):
            b = bits_v[pl.ds(j * 16, 16)]
            g = base + j * 16 + iota
            ev = jnp.where(g < _N, g, _PAD_EV)
            sel = (b > bst) | ((b == bst) & (ev <= ist))
            plsc.store_compressed(candb_v.at[pl.ds(m, 16)], b, mask=sel)
            plsc.store_compressed(cande_v.at[pl.ds(m, 16)], ev, mask=sel)
            return m + jnp.sum(sel.astype(jnp.int32))

        m = lax.fori_loop(0, _VR, compact_body, jnp.int32(0))

        my_v[...] = jnp.full((16,), m, jnp.int32)
        pltpu.sync_copy(my_v, sh_cnt.at[pl.ds(tid * 16, 16)])
        plsc.subcore_barrier()
        pltpu.sync_copy(sh_cnt, cnts_v)
        cnts = plsc.load_gather(cnts_v, [iota * 16])
        off = jnp.sum(jnp.where(iota < tid, cnts, 0))

        for c in range(10):
            for v in range(8):
                j = 128 * c + 16 * v + iota
                pos = jnp.where(j < jnp.full((16,), m), off + j, 1024 + tid)
                pos2_v[c, pl.ds(16 * v, 16)] = pos
        cps = []
        for c in range(10):
            cps.append(pltpu.async_copy(
                candb_v.at[pl.ds(128 * c, 128)],
                sh_candb.at[pos2_v.at[c]], sem_b))
            cps.append(pltpu.async_copy(
                cande_v.at[pl.ds(128 * c, 128)],
                sh_cande.at[pos2_v.at[c]], sem_b))
        for cp in cps:
            cp.wait()
        plsc.subcore_barrier()

        pltpu.sync_copy(sh_candb.at[pl.ds(0, 1024)], call_b)
        pltpu.sync_copy(sh_cande.at[pl.ds(0, 1024)], call_e)
        v62 = call_b[pl.ds(992, 16)]
        call_b[pl.ds(992, 16)] = jnp.where(iota < 8, v62, -1)
        call_b[pl.ds(1008, 16)] = jnp.full((16,), -1, jnp.int32)

        q0 = tid * 64
        for o in range(4):
            a_b = call_b[pl.ds(q0 + 16 * o, 16)]
            a_e = call_e[pl.ds(q0 + 16 * o, 16)]

            def pair_body(j, acc, a_b=a_b, a_e=a_e):
                b_b = call_b[pl.ds(j * 16, 16)]
                b_e = call_e[pl.ds(j * 16, 16)]
                for r in range(16):
                    ridx = (iota + r) % 16
                    bb = b_b[ridx]
                    be = b_e[ridx]
                    gt = bb > a_b
                    tie = (bb == a_b) & (be < a_e)
                    acc = acc + jnp.where(gt | tie, 1, 0)
                return acc

            acc = lax.fori_loop(0, 64, pair_body, iota * 0)
            q = q0 + 16 * o + iota
            rpos_v[0, pl.ds(16 * o, 16)] = jnp.where(
                q < _K, acc, 1024 + tid)
            rval_v[pl.ds(16 * o, 16)] = a_e
        pltpu.async_copy(rval_v, sh_sorted.at[rpos_v.at[0]], sem_c).wait()
        plsc.subcore_barrier()

        @pl.when(wid < _GT)
        def _():
            pltpu.sync_copy(sh_sorted.at[pl.ds(wid * _GR, _GR)],
                            ridx_v.at[pl.ds(0, _GR)])
            cpe = pltpu.async_copy(emb.at[ridx_v.at[pl.ds(0, _GR)]],
                                   em_rows, sem_a)

            def lg_idx(t, _):
                fpos = t * 16 + iota
                r = fpos // _C
                col = fpos - r * _C
                rid = plsc.load_gather(ridx_v, [jnp.minimum(r, 47)])
                lgi_v[pl.ds(t * 16, 16)] = jnp.where(
                    fpos < _LGW, rid * _CP + col, 0)
                return 0

            lax.fori_loop(0, (_LGCH * 128) // 16, lg_idx, 0)

            def bb_idx(t, _):
                fpos = t * 16 + iota
                r = fpos >> 2
                col = fpos & 3
                rid = plsc.load_gather(ridx_v, [jnp.minimum(r, 47)])
                bbi_v[pl.ds(t * 16, 16)] = jnp.where(
                    fpos < _BBW, rid * 4 + col, 0)
                return 0

            lax.fori_loop(0, (_BBCH * 128) // 16, bb_idx, 0)

            cpl = [pltpu.async_copy(lgf.at[lgi_v.at[pl.ds(128 * c, 128)]],
                                    lgd_v.at[pl.ds(128 * c, 128)], sem_b)
                   for c in range(_LGCH)]
            cpb = [pltpu.async_copy(bbf.at[bbi_v.at[pl.ds(128 * c, 128)]],
                                    bbd_v.at[pl.ds(128 * c, 128)], sem_c)
                   for c in range(_BBCH)]
            for cp in cpl:
                cp.wait()
            for cp in cpb:
                cp.wait()
            cpe.wait()
            pltpu.sync_copy(em_rows, oe.at[pl.ds(wid * _GR, _GR)])
            pltpu.sync_copy(lgd_v.at[pl.ds(0, _LGW)],
                            ol.at[pl.ds(wid * _LGW, _LGW)])
            pltpu.sync_copy(bbd_v.at[pl.ds(0, _BBW)],
                            ob.at[pl.ds(wid * _BBW, _BBW)])

    return sc_kernel


_TCROWS = 256


def _pad_tc_kernel(p_ref, l_ref, po_ref, lo_ref):
    zp = jnp.zeros((_TCROWS, _CP - _C), jnp.float32)
    po_ref[...] = jnp.concatenate([p_ref[...], zp], axis=1)
    lo_ref[...] = jnp.concatenate([l_ref[...], zp], axis=1)


def _pad_on_tc(cls_probs, cls_logits):
    grid = (_N + _TCROWS - 1) // _TCROWS
    return pl.pallas_call(
        _pad_tc_kernel,
        grid=(grid,),
        in_specs=[
            pl.BlockSpec((_TCROWS, _C), lambda i: (i, 0)),
            pl.BlockSpec((_TCROWS, _C), lambda i: (i, 0)),
        ],
        out_specs=[
            pl.BlockSpec((_TCROWS, _CP), lambda i: (i, 0)),
            pl.BlockSpec((_TCROWS, _CP), lambda i: (i, 0)),
        ],
        out_shape=[
            jax.ShapeDtypeStruct((_N, _CP), jnp.float32),
            jax.ShapeDtypeStruct((_N, _CP), jnp.float32),
        ],
    )(cls_probs, cls_logits)


def kernel(cls_probs, bboxes_cxcywh, cls_logits, embeds, image_size):
    probs_p, logits_p = _pad_on_tc(cls_probs, cls_logits)
    sc = _build_sc_kernel()
    ob, ol, oe = sc(
        probs_p.reshape(-1),
        bboxes_cxcywh.reshape(-1),
        logits_p.reshape(-1),
        embeds,
    )
    return (ob.reshape(_K, 4), ol.reshape(_K, _C), oe, image_size)


# speedup vs baseline: 1.4998x; 1.3701x over previous
import functools

import jax
import jax.numpy as jnp
from jax import lax
from jax.experimental import pallas as pl
from jax.experimental.pallas import tpu as pltpu
from jax.experimental.pallas import tpu_sc as plsc

_N = 20000
_C = 81
_D = 256
_K = 1000
_SCORE_COL = 80

_NT = 16
_NP = 1280
_VR = _NP // 16
_PAD_EV = 65535

_GT = 25
_GR = 40
_LGW = _GR * _C
_CP = 128
_LGCH = 26
_BBW = _GR * 4
_BBCH = 2


def _build_sc_kernel():
    mesh = plsc.VectorSubcoreMesh(core_axis_name="c", subcore_axis_name="s")

    @functools.partial(
        pl.kernel,
        mesh=mesh,
        compiler_params=pltpu.CompilerParams(needs_layout_passes=False),
        out_type=(
            jax.ShapeDtypeStruct((_K * 4,), jnp.float32),
            jax.ShapeDtypeStruct((_K * _C,), jnp.float32),
            jax.ShapeDtypeStruct((_K, _D), jnp.float32),
        ),
        scratch_types=[
            pltpu.VMEM((_NP,), jnp.float32),
            pltpu.VMEM((_NP,), jnp.int32),
            pltpu.VMEM((256,), jnp.int32),
            pltpu.VMEM((4096,), jnp.int32),
            pltpu.VMEM((_NP + 16,), jnp.int32),
            pltpu.VMEM((_NP + 16,), jnp.int32),
            pltpu.VMEM((16,), jnp.int32),
            pltpu.VMEM((256,), jnp.int32),
            pltpu.VMEM((10, 128), jnp.int32),
            pltpu.VMEM((1024,), jnp.int32),
            pltpu.VMEM((1024,), jnp.int32),
            pltpu.VMEM((1, 64), jnp.int32),
            pltpu.VMEM((64,), jnp.int32),
            pltpu.VMEM((48,), jnp.int32),
            pltpu.VMEM((_GR, _D), jnp.float32),
            pltpu.VMEM((_LGCH * 128 + 16,), jnp.int32),
            pltpu.VMEM((_LGCH * 128,), jnp.float32),
            pltpu.VMEM((_BBCH * 128,), jnp.int32),
            pltpu.VMEM((_BBCH * 128,), jnp.float32),
            pltpu.VMEM_SHARED((4096,), jnp.int32),
            pltpu.VMEM_SHARED((256,), jnp.int32),
            pltpu.VMEM_SHARED((1040,), jnp.int32),
            pltpu.VMEM_SHARED((1040,), jnp.int32),
            pltpu.VMEM_SHARED((1040,), jnp.int32),
            pltpu.SemaphoreType.DMA,
            pltpu.SemaphoreType.DMA,
            pltpu.SemaphoreType.DMA,
        ],
    )
    def sc_kernel(pf, bbf, lgf, emb, ob, ol, oe,
                  sval_v, bits_v, hist_v, hall_v, candb_v, cande_v,
                  my_v, cnts_v, pos2_v, call_b, call_e, rpos_v, rval_v,
                  ridx_v, em_rows, lgi_v, lgd_v, bbi_v, bbd_v,
                  sh_hist, sh_cnt, sh_candb, sh_cande, sh_sorted,
                  sem_a, sem_b, sem_c):
        cid = lax.axis_index("c")
        tid = lax.axis_index("s")
        wid = cid * _NT + tid
        iota = lax.iota(jnp.int32, 16)
        ones = jnp.full((16,), 1, jnp.int32)
        base = tid * _NP

        @pl.when(tid < _NT - 1)
        def _():
            pltpu.sync_copy(pf.at[pl.ds(base, _NP)], sval_v)

        @pl.when(tid == _NT - 1)
        def _():
            pltpu.sync_copy(pf.at[pl.ds(base, _N - (_NT - 1) * _NP)],
                            sval_v.at[pl.ds(0, _N - (_NT - 1) * _NP)])

        def p0_bits(j, _):
            s = sval_v[pl.ds(j * 16, 16)]
            b = plsc.bitcast(s, jnp.int32)
            g = base + j * 16 + iota
            bits_v[pl.ds(j * 16, 16)] = jnp.where(g < _N, b, 0)
            return 0

        lax.fori_loop(0, _VR, p0_bits, 0)

        R = jnp.int32(_K)
        fp = jnp.int32(0)
        d5 = jnp.int32(0)
        istar = jnp.int32(0)
        bstar = jnp.int32(0)

        for p in range(6):
            def zero_h(t, _):
                hist_v[pl.ds(t * 16, 16)] = iota * 0
                return 0

            lax.fori_loop(0, 16, zero_h, 0)

            fpb = fp

            def hist_body(j, _, p=p, fpb=fpb, R=R, d5=d5, bstar=bstar):
                b = bits_v[pl.ds(j * 16, 16)]
                g = base + j * 16 + iota
                ev = jnp.where(g < _N, g, _PAD_EV)
                if p == 0:
                    cand = jnp.full((16,), True)
                    dig = (b >> 24) & 255
                elif p == 1:
                    cand = (b >> 24) == jnp.full((16,), fpb >> 24)
                    dig = (b >> 16) & 255
                elif p == 2:
                    cand = (b >> 16) == jnp.full((16,), fpb >> 16)
                    dig = (b >> 8) & 255
                elif p == 3:
                    cand = (b >> 8) == jnp.full((16,), fpb >> 8)
                    dig = b & 255
                elif p == 4:
                    cand = b == jnp.full((16,), bstar)
                    dig = (ev >> 8) & 255
                else:
                    cand = ((b == jnp.full((16,), bstar))
                            & ((ev >> 8) == jnp.full((16,), d5)))
                    dig = ev & 255
                plsc.addupdate_scatter(hist_v, [dig], ones, mask=cand)
                return 0

            lax.fori_loop(0, _VR, hist_body, 0)

            plsc.subcore_barrier()
            pltpu.sync_copy(hist_v, sh_hist.at[pl.ds(tid * 256, 256)])
            plsc.subcore_barrier()
            pltpu.sync_copy(sh_hist, hall_v)

            hs = []
            for v in range(16):
                def red_body(r, acc, v=v):
                    return acc + hall_v[pl.ds(r * 256 + v * 16, 16)]

                hs.append(lax.fori_loop(0, 16, red_body, iota * 0))

            tv = [jnp.sum(h) for h in hs]
            if p < 4:
                suf = [jnp.int32(0)] * 17
                for v in range(15, -1, -1):
                    suf[v] = suf[v + 1] + tv[v]
                dstar = jnp.int32(-1)
                slanes = []
                for v in range(16):
                    s_lane = (jnp.full((16,), suf[v + 1])
                              + lax.rev(plsc.cumsum(lax.rev(hs[v], (0,))),
                                        (0,)))
                    slanes.append(s_lane)
                    digv = iota + 16 * v
                    cd = jnp.where(s_lane >= jnp.full((16,), R), digv, -1)
                    dstar = jnp.maximum(dstar, jnp.max(cd))
                hd = jnp.int32(0)
                sd = jnp.int32(0)
                for v in range(16):
                    selv = (iota + 16 * v) == jnp.full((16,), dstar)
                    hd = hd + jnp.sum(jnp.where(selv, hs[v], 0))
                    sd = sd + jnp.sum(jnp.where(selv, slanes[v], 0))
                R = R - (sd - hd)
                fp = fp | (dstar << (24 - 8 * p))
                if p == 3:
                    bstar = fp
            else:
                pre = [jnp.int32(0)] * 17
                for v in range(16):
                    pre[v + 1] = pre[v] + tv[v]
                dstar = jnp.int32(9999)
                planes = []
                for v in range(16):
                    p_lane = jnp.full((16,), pre[v]) + plsc.cumsum(hs[v])
                    planes.append(p_lane)
                    digv = iota + 16 * v
                    cd = jnp.where(p_lane >= jnp.full((16,), R), digv, 9999)
                    dstar = jnp.minimum(dstar, jnp.min(cd))
                hd = jnp.int32(0)
                pd = jnp.int32(0)
                for v in range(16):
                    selv = (iota + 16 * v) == jnp.full((16,), dstar)
                    hd = hd + jnp.sum(jnp.where(selv, hs[v], 0))
                    pd = pd + jnp.sum(jnp.where(selv, planes[v], 0))
                R = R - (pd - hd)
                if p == 4:
                    d5 = dstar
                else:
                    istar = (d5 << 8) | dstar

        bst = jnp.full((16,), bstar)
        ist = jnp.full((16,), istar)

        def compact_body(j, m):
            b = bits_v[pl.ds(j * 16, 16)]
            g = base + j * 16 + iota
            ev = jnp.where(g < _N, g, _PAD_EV)
            sel = (b > bst) | ((b == bst) & (ev <= ist))
            plsc.store_compressed(candb_v.at[pl.ds(m, 16)], b, mask=sel)
            plsc.store_compressed(cande_v.at[pl.ds(m, 16)], ev, mask=sel)
            return m + jnp.sum(sel.astype(jnp.int32))

        m = lax.fori_loop(0, _VR, compact_body, jnp.int32(0))

        my_v[...] = jnp.full((16,), m, jnp.int32)
        pltpu.sync_copy(my_v, sh_cnt.at[pl.ds(tid * 16, 16)])
        plsc.subcore_barrier()
        pltpu.sync_copy(sh_cnt, cnts_v)
        cnts = plsc.load_gather(cnts_v, [iota * 16])
        off = jnp.sum(jnp.where(iota < tid, cnts, 0))

        for c in range(10):
            for v in range(8):
                j = 128 * c + 16 * v + iota
                pos = jnp.where(j < jnp.full((16,), m), off + j, 1024 + tid)
                pos2_v[c, pl.ds(16 * v, 16)] = pos
        cps = []
        for c in range(10):
            cps.append(pltpu.async_copy(
                candb_v.at[pl.ds(128 * c, 128)],
                sh_candb.at[pos2_v.at[c]], sem_b))
            cps.append(pltpu.async_copy(
                cande_v.at[pl.ds(128 * c, 128)],
                sh_cande.at[pos2_v.at[c]], sem_b))
        for cp in cps:
            cp.wait()
        plsc.subcore_barrier()

        pltpu.sync_copy(sh_candb.at[pl.ds(0, 1024)], call_b)
        pltpu.sync_copy(sh_cande.at[pl.ds(0, 1024)], call_e)
        v62 = call_b[pl.ds(992, 16)]
        call_b[pl.ds(992, 16)] = jnp.where(iota < 8, v62, -1)
        call_b[pl.ds(1008, 16)] = jnp.full((16,), -1, jnp.int32)

        q0 = tid * 64
        for o in range(4):
            a_b = call_b[pl.ds(q0 + 16 * o, 16)]
            a_e = call_e[pl.ds(q0 + 16 * o, 16)]

            def pair_body(j, acc, a_b=a_b, a_e=a_e):
                b_b = call_b[pl.ds(j * 16, 16)]
                b_e = call_e[pl.ds(j * 16, 16)]
                for r in range(16):
                    ridx = (iota + r) % 16
                    bb = b_b[ridx]
                    be = b_e[ridx]
                    gt = bb > a_b
                    tie = (bb == a_b) & (be < a_e)
                    acc = acc + jnp.where(gt | tie, 1, 0)
                return acc

            acc = lax.fori_loop(0, 64, pair_body, iota * 0)
            q = q0 + 16 * o + iota
            rpos_v[0, pl.ds(16 * o, 16)] = jnp.where(
                q < _K, acc, 1024 + tid)
            rval_v[pl.ds(16 * o, 16)] = a_e
        pltpu.async_copy(rval_v, sh_sorted.at[rpos_v.at[0]], sem_c).wait()
        plsc.subcore_barrier()

        @pl.when(wid < _GT)
        def _():
            pltpu.sync_copy(sh_sorted.at[pl.ds(wid * _GR, _GR)],
                            ridx_v.at[pl.ds(0, _GR)])
            cpe = pltpu.async_copy(emb.at[ridx_v.at[pl.ds(0, _GR)]],
                                   em_rows, sem_a)

            def lg_idx(t, _):
                fpos = t * 16 + iota
                r = fpos // _C
                col = fpos - r * _C
                rid = plsc.load_gather(ridx_v, [jnp.minimum(r, 47)])
                lgi_v[pl.ds(t * 16, 16)] = jnp.where(
                    fpos < _LGW, rid * _CP + col, 0)
                return 0

            lax.fori_loop(0, (_LGCH * 128) // 16, lg_idx, 0)

            def bb_idx(t, _):
                fpos = t * 16 + iota
                r = fpos >> 2
                col = fpos & 3
                rid = plsc.load_gather(ridx_v, [jnp.minimum(r, 47)])
                bbi_v[pl.ds(t * 16, 16)] = jnp.where(
                    fpos < _BBW, rid * _CP + col, 0)
                return 0

            lax.fori_loop(0, (_BBCH * 128) // 16, bb_idx, 0)

            cpl = [pltpu.async_copy(lgf.at[lgi_v.at[pl.ds(128 * c, 128)]],
                                    lgd_v.at[pl.ds(128 * c, 128)], sem_b)
                   for c in range(_LGCH)]
            cpb = [pltpu.async_copy(bbf.at[bbi_v.at[pl.ds(128 * c, 128)]],
                                    bbd_v.at[pl.ds(128 * c, 128)], sem_c)
                   for c in range(_BBCH)]
            for cp in cpl:
                cp.wait()
            for cp in cpb:
                cp.wait()
            cpe.wait()
            pltpu.sync_copy(em_rows, oe.at[pl.ds(wid * _GR, _GR)])
            pltpu.sync_copy(lgd_v.at[pl.ds(0, _LGW)],
                            ol.at[pl.ds(wid * _LGW, _LGW)])
            pltpu.sync_copy(bbd_v.at[pl.ds(0, _BBW)],
                            ob.at[pl.ds(wid * _BBW, _BBW)])

    return sc_kernel


_TCROWS = 2048


def _prep_tc_kernel(p_ref, l_ref, b_ref, s_ref, lo_ref, bo_ref):
    s_ref[...] = p_ref[...][:, _SCORE_COL]
    zl = jnp.zeros((_TCROWS, _CP - _C), jnp.float32)
    lo_ref[...] = jnp.concatenate([l_ref[...], zl], axis=1)
    zb = jnp.zeros((_TCROWS, _CP - 4), jnp.float32)
    bo_ref[...] = jnp.concatenate([b_ref[...], zb], axis=1)


def _prep_on_tc(cls_probs, cls_logits, bboxes):
    grid = (_N + _TCROWS - 1) // _TCROWS
    return pl.pallas_call(
        _prep_tc_kernel,
        grid=(grid,),
        in_specs=[
            pl.BlockSpec((_TCROWS, _C), lambda i: (i, 0)),
            pl.BlockSpec((_TCROWS, _C), lambda i: (i, 0)),
            pl.BlockSpec((_TCROWS, 4), lambda i: (i, 0)),
        ],
        out_specs=[
            pl.BlockSpec((_TCROWS,), lambda i: (i,)),
            pl.BlockSpec((_TCROWS, _CP), lambda i: (i, 0)),
            pl.BlockSpec((_TCROWS, _CP), lambda i: (i, 0)),
        ],
        out_shape=[
            jax.ShapeDtypeStruct((_N,), jnp.float32),
            jax.ShapeDtypeStruct((_N, _CP), jnp.float32),
            jax.ShapeDtypeStruct((_N, _CP), jnp.float32),
        ],
    )(cls_probs, cls_logits, bboxes)


def kernel(cls_probs, bboxes_cxcywh, cls_logits, embeds, image_size):
    scores, logits_p, bb_p = _prep_on_tc(cls_probs, cls_logits, bboxes_cxcywh)
    sc = _build_sc_kernel()
    ob, ol, oe = sc(
        scores,
        bb_p.reshape(-1),
        logits_p.reshape(-1),
        embeds,
    )
    return (ob.reshape(_K, 4), ol.reshape(_K, _C), oe, image_size)
